# baseline (speedup 1.0000x reference)
import functools

import jax
import jax.numpy as jnp
from jax import lax
from jax.experimental import pallas as pl
from jax.experimental.pallas import tpu as pltpu
from jax.experimental.pallas import tpu_sc as plsc

EMB_DIM = 128

NC = 2
NS = 16
NW = NC * NS

NBUF = 4


@jax.jit
def _gather_rows(idx, table):
    batch, fields = idx.shape
    s_per_w = batch // NW
    mesh = plsc.VectorSubcoreMesh(core_axis_name="c", subcore_axis_name="s")

    @functools.partial(
        pl.kernel,
        mesh=mesh,
        compiler_params=pltpu.CompilerParams(use_tc_tiling_on_sc=True),
        out_type=jax.ShapeDtypeStruct((batch, fields, EMB_DIM), jnp.float32),
        scratch_types=[
            pltpu.VMEM((s_per_w, fields), jnp.int32),
            pltpu.VMEM((fields, EMB_DIM), jnp.float32),
            pltpu.VMEM((fields, EMB_DIM), jnp.float32),
            pltpu.VMEM((fields, EMB_DIM), jnp.float32),
            pltpu.VMEM((fields, EMB_DIM), jnp.float32),
            pltpu.SemaphoreType.DMA,
            pltpu.SemaphoreType.DMA,
            pltpu.SemaphoreType.DMA,
            pltpu.SemaphoreType.DMA,
        ],
    )
    def k(idx_hbm, table_hbm, out_hbm, idx_v, b0, b1, b2, b3, s0, s1, s2, s3):
        bufs = (b0, b1, b2, b3)
        sems = (s0, s1, s2, s3)
        wid = lax.axis_index("s") * NC + lax.axis_index("c")
        sample0 = wid * s_per_w

        pltpu.sync_copy(idx_hbm.at[pl.ds(sample0, s_per_w)], idx_v)

        def gather(c, b):
            pltpu.async_copy(table_hbm.at[idx_v.at[c]], bufs[b], sems[b])

        for c in range(NBUF - 1):
            gather(c, c)

        def body(g, carry):
            for b in range(NBUF):
                c = g * NBUF + b
                pltpu.make_async_copy(table_hbm.at[idx_v.at[c]], bufs[b],
                                      sems[b]).wait()

                @pl.when(c + NBUF - 1 < s_per_w)
                def _():
                    gather(c + NBUF - 1, (b + NBUF - 1) % NBUF)

                pltpu.sync_copy(bufs[b], out_hbm.at[sample0 + c])
            return carry

        lax.fori_loop(0, s_per_w // NBUF, body, 0)

    return k(idx, table)


def kernel(input, table):
    return _gather_rows(input.astype(jnp.int32), table)


# speedup vs baseline: 1.8191x; 1.8191x over previous
import functools

import jax
import jax.numpy as jnp
from jax import lax
from jax.experimental import pallas as pl
from jax.experimental.pallas import tpu as pltpu
from jax.experimental.pallas import tpu_sc as plsc

EMB_DIM = 128

NC = 2
NS = 16
NW = NC * NS

NBUF = 4


@jax.jit
def _gather_rows(idx_t, table):
    fields, batch = idx_t.shape
    s_per_w = batch // NW
    mesh = plsc.VectorSubcoreMesh(core_axis_name="c", subcore_axis_name="s")

    @functools.partial(
        pl.kernel,
        mesh=mesh,
        out_type=jax.ShapeDtypeStruct((fields, batch, EMB_DIM), jnp.float32),
        scratch_types=[
            pltpu.VMEM((fields, s_per_w), jnp.int32),
            pltpu.VMEM((s_per_w, EMB_DIM), jnp.float32),
            pltpu.VMEM((s_per_w, EMB_DIM), jnp.float32),
            pltpu.VMEM((s_per_w, EMB_DIM), jnp.float32),
            pltpu.VMEM((s_per_w, EMB_DIM), jnp.float32),
            pltpu.SemaphoreType.DMA,
            pltpu.SemaphoreType.DMA,
            pltpu.SemaphoreType.DMA,
            pltpu.SemaphoreType.DMA,
        ],
    )
    def k(idx_hbm, table_hbm, out_hbm, idx_v, b0, b1, b2, b3, s0, s1, s2, s3):
        bufs = (b0, b1, b2, b3)
        sems = (s0, s1, s2, s3)
        wid = lax.axis_index("s") * NC + lax.axis_index("c")
        sample0 = wid * s_per_w

        pltpu.sync_copy(idx_hbm.at[:, pl.ds(sample0, s_per_w)], idx_v)

        def gather(c, b):
            pltpu.async_copy(table_hbm.at[idx_v.at[c]], bufs[b], sems[b])

        for c in range(NBUF - 1):
            gather(c, c)

        def body(g, carry):
            for b in range(NBUF):
                c = g * NBUF + b
                pltpu.make_async_copy(table_hbm.at[idx_v.at[c]], bufs[b],
                                      sems[b]).wait()

                @pl.when(c + NBUF - 1 < fields)
                def _():
                    gather(c + NBUF - 1, (b + NBUF - 1) % NBUF)

                pltpu.sync_copy(bufs[b],
                                out_hbm.at[c, pl.ds(sample0, s_per_w)])
            return carry

        lax.fori_loop(0, fields // NBUF, body, 0)

    return k(idx_t, table)


def kernel(input, table):
    out_t = _gather_rows(input.T.astype(jnp.int32), table)
    return out_t.transpose(1, 0, 2)
